# Initial kernel scaffold; baseline (speedup 1.0000x reference)
import functools

import jax
import jax.numpy as jnp
from jax import lax
from jax.experimental import pallas as pl
from jax.experimental.pallas import tpu as pltpu
from jax.experimental.pallas import tpu_sc as plsc

N_CELL, E_CELL, N_TIS, E_TIS, D, H, DH = 10000, 320000, 1000, 8000, 128, 8, 16
NC, NS = 2, 16
NW = NC * NS
CG = 80
F32 = jnp.float32


def _dense_body(act, x_ref, w_ref, b_ref, o_ref):
    y = jnp.dot(x_ref[...], w_ref[...], preferred_element_type=F32)
    if b_ref is not None:
        y = y + b_ref[...]
    if act == "relu":
        y = jnp.maximum(y, 0.0)
    o_ref[...] = y


def tc_dense(x, w, b=None, act=None, block=2000):
    n, _ = x.shape
    k = w.shape[1]
    grid = max(n // block, 1)
    blk = n // grid
    if b is None:
        body = lambda x_ref, w_ref, o_ref: _dense_body(act, x_ref, w_ref, None, o_ref)
        in_specs = [pl.BlockSpec((blk, x.shape[1]), lambda i: (i, 0)),
                    pl.BlockSpec(w.shape, lambda i: (0, 0))]
        args = (x, w)
    else:
        body = functools.partial(_dense_body, act)
        in_specs = [pl.BlockSpec((blk, x.shape[1]), lambda i: (i, 0)),
                    pl.BlockSpec(w.shape, lambda i: (0, 0)),
                    pl.BlockSpec(b.shape, lambda i: (0, 0))]
        args = (x, w, b)
    return pl.pallas_call(
        body, grid=(grid,),
        in_specs=in_specs,
        out_specs=pl.BlockSpec((blk, k), lambda i: (i, 0)),
        out_shape=jax.ShapeDtypeStruct((n, k), F32),
    )(*args)


def _pre_gat_body(x_ref, w_ref, asrc_ref, adst_ref, h_ref, esd_ref, pmax_ref):
    h = jnp.dot(x_ref[...], w_ref[...], preferred_element_type=F32)
    h_ref[...] = h
    r = lax.broadcasted_iota(jnp.int32, (D, H), 0) // DH
    c = lax.broadcasted_iota(jnp.int32, (D, H), 1)
    pb = jnp.where(r == c, 1.0, 0.0).astype(F32)
    es = jnp.dot(h * asrc_ref[...], pb, preferred_element_type=F32)
    ed = jnp.dot(h * adst_ref[...], pb, preferred_element_type=F32)
    esd = jnp.concatenate([es, ed], axis=1)
    esd_ref[...] = esd
    pm = jnp.broadcast_to(jnp.max(esd, axis=0, keepdims=True), (8, 2 * H))

    @pl.when(pl.program_id(0) == 0)
    def _():
        pmax_ref[...] = pm

    @pl.when(pl.program_id(0) != 0)
    def _():
        pmax_ref[...] = jnp.maximum(pmax_ref[...], pm)


def tc_pre_gat(x, w, asrc, adst):
    n = x.shape[0]
    grid, blk = 5, n // 5
    return pl.pallas_call(
        _pre_gat_body, grid=(grid,),
        in_specs=[pl.BlockSpec((blk, D), lambda i: (i, 0)),
                  pl.BlockSpec((D, D), lambda i: (0, 0)),
                  pl.BlockSpec((1, D), lambda i: (0, 0)),
                  pl.BlockSpec((1, D), lambda i: (0, 0))],
        out_specs=[pl.BlockSpec((blk, D), lambda i: (i, 0)),
                   pl.BlockSpec((blk, 2 * H), lambda i: (i, 0)),
                   pl.BlockSpec((8, 2 * H), lambda i: (0, 0))],
        out_shape=[jax.ShapeDtypeStruct((n, D), F32),
                   jax.ShapeDtypeStruct((n, 2 * H), F32),
                   jax.ShapeDtypeStruct((8, 2 * H), F32)],
    )(x, w, asrc, adst)


def _post_gat_body(o2_ref, s2_ref, b_ref, res_ref, g_ref, bln_ref, o_ref):
    o = o2_ref[0] + o2_ref[1]
    s = s2_ref[0] + s2_ref[1]
    s8 = s[:, :H]
    r = lax.broadcasted_iota(jnp.int32, (H, D), 0)
    c = lax.broadcasted_iota(jnp.int32, (H, D), 1) // DH
    krep = jnp.where(r == c, 1.0, 0.0).astype(F32)
    srep = jnp.dot(s8, krep, preferred_element_type=F32)
    val = o / (srep + 1e-16) + b_ref[...] + res_ref[...]
    mu = jnp.mean(val, axis=1, keepdims=True)
    dv = val - mu
    var = jnp.mean(dv * dv, axis=1, keepdims=True)
    ln = dv / jnp.sqrt(var + 1e-5) * g_ref[...] + bln_ref[...]
    o_ref[...] = jnp.maximum(ln, 0.0)


def tc_post_gat(o2, s2, b, res, g, bln):
    n = res.shape[0]
    grid, blk = 5, n // 5
    return pl.pallas_call(
        _post_gat_body, grid=(grid,),
        in_specs=[pl.BlockSpec((2, blk, D), lambda i: (0, i, 0)),
                  pl.BlockSpec((2, blk, 2 * H), lambda i: (0, i, 0)),
                  pl.BlockSpec((1, D), lambda i: (0, 0)),
                  pl.BlockSpec((blk, D), lambda i: (i, 0)),
                  pl.BlockSpec((1, D), lambda i: (0, 0)),
                  pl.BlockSpec((1, D), lambda i: (0, 0))],
        out_specs=pl.BlockSpec((blk, D), lambda i: (i, 0)),
        out_shape=jax.ShapeDtypeStruct((n, D), F32),
    )(o2, s2, b, res, g, bln)


def _count_body(mode, nseg, idx_ref, acc_ref, out_ref):
    @pl.when(pl.program_id(0) == 0)
    def _():
        acc_ref[...] = jnp.zeros_like(acc_ref)

    idx = idx_ref[...]
    cols = lax.broadcasted_iota(jnp.int32, (idx.shape[0], nseg), 1)
    m = jnp.where(idx == cols, 1.0, 0.0).astype(F32)
    acc_ref[...] += jnp.broadcast_to(jnp.sum(m, axis=0, keepdims=True), (8, nseg))

    @pl.when(pl.program_id(0) == pl.num_programs(0) - 1)
    def _():
        deg = acc_ref[...]
        if mode == "dinv":
            out_ref[...] = lax.rsqrt(deg + 1.0)
        else:
            out_ref[...] = 1.0 / jnp.maximum(deg, 1.0)


def tc_count(idx_col, nseg, mode):
    n = idx_col.shape[0]
    grid, blk = 4, n // 4
    acc, out = pl.pallas_call(
        functools.partial(_count_body, mode, nseg), grid=(grid,),
        in_specs=[pl.BlockSpec((blk, 1), lambda i: (i, 0))],
        out_specs=[pl.BlockSpec((8, nseg), lambda i: (0, 0)),
                   pl.BlockSpec((8, nseg), lambda i: (0, 0))],
        out_shape=[jax.ShapeDtypeStruct((8, nseg), F32),
                   jax.ShapeDtypeStruct((8, nseg), F32)],
    )(idx_col)
    del acc
    return out


def _post_gcn_body(a2_ref, xw_ref, dv_ref, b_ref, g_ref, bln_ref, res_ref, o_ref):
    xw = xw_ref[...]
    dv = dv_ref[...]
    t2 = a2_ref[0] + a2_ref[1] + dv * dv * xw + b_ref[...]
    mu = jnp.mean(t2, axis=1, keepdims=True)
    d_ = t2 - mu
    var = jnp.mean(d_ * d_, axis=1, keepdims=True)
    ln = d_ / jnp.sqrt(var + 1e-5) * g_ref[...] + bln_ref[...]
    o_ref[...] = jnp.maximum(ln, 0.0) + res_ref[...]


def tc_post_gcn(a2, xw, dinv_col, b, g, bln, res):
    n = xw.shape[0]
    return pl.pallas_call(
        _post_gcn_body,
        out_shape=jax.ShapeDtypeStruct((n, D), F32),
    )(a2, xw, dinv_col, b, g, bln, res)


def _precross_body(x_ref, wo_ref, bo_ref, wc_ref, wu_ref, cf_ref, cwc_ref, cwu_ref):
    cf = jnp.dot(x_ref[...], wo_ref[...], preferred_element_type=F32) + bo_ref[...]
    cf_ref[...] = cf
    cwc_ref[...] = jnp.dot(cf, wc_ref[...], preferred_element_type=F32)
    cwu_ref[...] = jnp.dot(cf, wu_ref[...], preferred_element_type=F32)


def tc_precross(x, wo, bo, wc, wu):
    n = x.shape[0]
    grid, blk = 5, n // 5
    wspec = pl.BlockSpec((D, D), lambda i: (0, 0))
    return pl.pallas_call(
        _precross_body, grid=(grid,),
        in_specs=[pl.BlockSpec((blk, D), lambda i: (i, 0)), wspec,
                  pl.BlockSpec((1, D), lambda i: (0, 0)), wspec, wspec],
        out_specs=[pl.BlockSpec((blk, D), lambda i: (i, 0))] * 3,
        out_shape=[jax.ShapeDtypeStruct((n, D), F32)] * 3,
    )(x, wo, bo, wc, wu)


def _cell_attn_body(cf_ref, cwc_ref, g1_ref, g2_ref, o_ref):
    score = jnp.sum(cwc_ref[...] * g1_ref[...], axis=1, keepdims=True) * (
        1.0 / jnp.sqrt(float(D)))
    alpha = 1.0 / (1.0 + jnp.exp(-score))
    o_ref[...] = cf_ref[...] + alpha * g2_ref[...]


def tc_cell_attn(cf, cwc, g1, g2):
    n = cf.shape[0]
    grid, blk = 5, n // 5
    spec = pl.BlockSpec((blk, D), lambda i: (i, 0))
    return pl.pallas_call(
        _cell_attn_body, grid=(grid,),
        in_specs=[spec] * 4,
        out_specs=spec,
        out_shape=jax.ShapeDtypeStruct((n, D), F32),
    )(cf, cwc, g1, g2)


def _heads_body(ca_ref, t_ref, agg_ref, rc_ref,
                pcw1_ref, pcb1_ref, pcw2_ref,
                ptw1_ref, ptb1_ref, ptw2_ref,
                fw1_ref, fb1_ref, fw2_ref, fb2_ref,
                cw1_ref, cb1_ref, cw2_ref, cb2_ref,
                kw1_ref, kb1_ref, kw2_ref, kb2_ref,
                lo_ref, co_ref):
    tis = t_ref[...] + (agg_ref[0] + agg_ref[1]) * rc_ref[...]

    def pool(x, w1, b1, w2):
        s = jnp.dot(jnp.tanh(jnp.dot(x, w1, preferred_element_type=F32) + b1),
                    w2, preferred_element_type=F32)
        s = s - jnp.max(s, axis=0, keepdims=True)
        a = jnp.exp(s)
        a = a / jnp.sum(a, axis=0, keepdims=True)
        return jnp.sum(a * x, axis=0, keepdims=True)

    cr = pool(ca_ref[...], pcw1_ref[...], pcb1_ref[...], pcw2_ref[...])
    tr = pool(tis, ptw1_ref[...], ptb1_ref[...], ptw2_ref[...])
    fused = jnp.broadcast_to(jnp.concatenate([cr, tr], axis=1), (8, 2 * D))
    f1 = jnp.maximum(jnp.dot(fused, fw1_ref[...], preferred_element_type=F32)
                     + fb1_ref[...], 0.0)
    f2 = jnp.maximum(jnp.dot(f1, fw2_ref[...], preferred_element_type=F32)
                     + fb2_ref[...], 0.0)
    c1 = jnp.maximum(jnp.dot(f2, cw1_ref[...], preferred_element_type=F32)
                     + cb1_ref[...], 0.0)
    lo_ref[...] = jnp.dot(c1, cw2_ref[...], preferred_element_type=F32) + cb2_ref[...]
    k1 = jnp.maximum(jnp.dot(f2, kw1_ref[...], preferred_element_type=F32)
                     + kb1_ref[...], 0.0)
    co_ref[...] = jnp.dot(k1, kw2_ref[...], preferred_element_type=F32) + kb2_ref[...]


def tc_heads(ca, t, agg, rc_col, pc, pt, fu, cl, ct):
    return pl.pallas_call(
        _heads_body,
        out_shape=[jax.ShapeDtypeStruct((8, 4), F32),
                   jax.ShapeDtypeStruct((8, 10), F32)],
    )(ca, t, agg, rc_col,
      pc["W1"], pc["b1"].reshape(1, D), pc["w2"].reshape(D, 1),
      pt["W1"], pt["b1"].reshape(1, D), pt["w2"].reshape(D, 1),
      fu["W1"], fu["b1"].reshape(1, D), fu["W2"], fu["b2"].reshape(1, D // 2),
      cl["W1"], cl["b1"].reshape(1, D // 4), cl["W2"], cl["b2"].reshape(1, 4),
      ct["W1"], ct["b1"].reshape(1, D // 2), ct["W2"], ct["b2"].reshape(1, 10))


_GDN = lax.GatherDimensionNumbers(
    offset_dims=(), collapsed_slice_dims=(0,), start_index_map=(0,))


def _take16(x, idx):
    return lax.gather(x, idx[:, None], _GDN, (1,),
                      mode=lax.GatherScatterMode.PROMISE_IN_BOUNDS)


def _zero_rows(ref, rows, lanes):
    z = jnp.zeros((16,), F32)
    n16 = lanes // 16

    def body(i, _):
        r = i // n16
        c = (i - r * n16) * 16
        ref[r, pl.ds(c, 16)] = z
        return 0

    lax.fori_loop(0, rows * n16, body, 0)


_SC_MESH = plsc.VectorSubcoreMesh(core_axis_name="c", subcore_axis_name="s")


def _gat_sc_body(esd_hbm, h_hbm, src_hbm, dst_hbm, gmax_hbm,
                 out_hbm, sacc_hbm,
                 sidx, didx, sb, tb, rb, wb, yb, gbuf, zb, zbs,
                 out_sh, s_sh):
    cid = lax.axis_index("c")
    sid = lax.axis_index("s")
    wid = sid * NC + cid
    rpt = N_CELL // NS

    _zero_rows(zb, 125, D)
    _zero_rows(zbs, rpt, 2 * H)
    for k in range(5):
        pltpu.sync_copy(zb, out_sh.at[pl.ds(sid * rpt + k * 125, 125)])
    pltpu.sync_copy(zbs, s_sh.at[pl.ds(sid * rpt, rpt)])
    plsc.subcore_barrier()

    pltpu.sync_copy(gmax_hbm, gbuf)
    gvec = gbuf[...]
    rot = jnp.remainder(lax.iota(jnp.int32, 16) + 8, 16)
    hsel = [jnp.full((16,), hh, jnp.int32) for hh in range(H)]
    ebase = wid * (E_CELL // NW)

    def chunk(j, _):
        base = ebase + j * CG
        pltpu.sync_copy(src_hbm.at[pl.ds(base, CG)], sidx)
        pltpu.sync_copy(dst_hbm.at[pl.ds(base, CG)], didx)
        pltpu.sync_copy(esd_hbm.at[sidx], sb)
        pltpu.sync_copy(esd_hbm.at[didx], tb)
        pltpu.sync_copy(h_hbm.at[sidx], rb)

        def edge(i, _):
            sv = sb[i]
            tv = tb[i]
            e = sv + _take16(tv, rot)
            e = jnp.maximum(e, 0.2 * e)
            w = jnp.exp(e - gvec)
            wb[i] = w
            for hh in range(H):
                wbc = _take16(w, hsel[hh])
                yb[i, pl.ds(hh * 16, 16)] = wbc * rb[i, pl.ds(hh * 16, 16)]
            return 0

        lax.fori_loop(0, CG, edge, 0)
        pltpu.sync_copy(wb, s_sh.at[didx], add=True)
        pltpu.sync_copy(yb, out_sh.at[didx], add=True)
        return 0

    lax.fori_loop(0, E_CELL // NW // CG, chunk, 0)
    plsc.subcore_barrier()
    pltpu.sync_copy(out_sh.at[pl.ds(sid * rpt, rpt)],
                    out_hbm.at[cid, pl.ds(sid * rpt, rpt)])
    pltpu.sync_copy(s_sh.at[pl.ds(sid * rpt, rpt)],
                    sacc_hbm.at[cid, pl.ds(sid * rpt, rpt)])


@functools.partial(
    pl.kernel, mesh=_SC_MESH,
    out_type=[jax.ShapeDtypeStruct((NC, N_CELL, D), F32),
              jax.ShapeDtypeStruct((NC, N_CELL, 2 * H), F32)],
    scratch_types=[
        pltpu.VMEM((CG,), jnp.int32), pltpu.VMEM((CG,), jnp.int32),
        pltpu.VMEM((CG, 2 * H), F32), pltpu.VMEM((CG, 2 * H), F32),
        pltpu.VMEM((CG, D), F32), pltpu.VMEM((CG, 2 * H), F32),
        pltpu.VMEM((CG, D), F32), pltpu.VMEM((16,), F32),
        pltpu.VMEM((125, D), F32), pltpu.VMEM((N_CELL // NS, 2 * H), F32),
        pltpu.VMEM_SHARED((N_CELL, D), F32),
        pltpu.VMEM_SHARED((N_CELL, 2 * H), F32),
    ])
def sc_gat(*args):
    _gat_sc_body(*args)


def _tile_split_1000(sid):
    return jnp.where(sid < 8, sid * 63, 504 + (sid - 8) * 62)


def _zero_spmem_1000(zb, sh, sid):
    base = _tile_split_1000(sid)

    @pl.when(sid < 8)
    def _():
        pltpu.sync_copy(zb, sh.at[pl.ds(base, 63)])

    @pl.when(sid >= 8)
    def _():
        pltpu.sync_copy(zb.at[pl.ds(0, 62)], sh.at[pl.ds(base, 62)])


def _copy_out_1000(sh, out_hbm, cid, sid):
    base = _tile_split_1000(sid)

    @pl.when(sid < 8)
    def _():
        pltpu.sync_copy(sh.at[pl.ds(base, 63)], out_hbm.at[cid, pl.ds(base, 63)])

    @pl.when(sid >= 8)
    def _():
        pltpu.sync_copy(sh.at[pl.ds(base, 62)], out_hbm.at[cid, pl.ds(base, 62)])


def _gcn_sc_body(xw_hbm, dinv_hbm, src_hbm, dst_hbm, agg_hbm,
                 sidx, didx, dvbuf, rb, yb, wc, zb, agg_sh):
    cid = lax.axis_index("c")
    sid = lax.axis_index("s")
    wid = sid * NC + cid

    _zero_rows(zb, 63, D)
    _zero_spmem_1000(zb, agg_sh, sid)
    pltpu.sync_copy(dinv_hbm, dvbuf)
    plsc.subcore_barrier()

    nchunk = E_TIS // CG
    for jj in range((nchunk + NW - 1) // NW):
        j = wid + NW * jj

        @pl.when(j < nchunk)
        def _():
            base = j * CG
            pltpu.sync_copy(src_hbm.at[pl.ds(base, CG)], sidx)
            pltpu.sync_copy(dst_hbm.at[pl.ds(base, CG)], didx)
            pltpu.sync_copy(xw_hbm.at[sidx], rb)
            for g in range(CG // 16):
                sv = sidx[pl.ds(g * 16, 16)]
                dv = didx[pl.ds(g * 16, 16)]
                w = (plsc.load_gather(dvbuf, [sv]) *
                     plsc.load_gather(dvbuf, [dv]))
                wc[pl.ds(g * 16, 16)] = w

            def edge(i, _):
                ws = wc[i]
                for hh in range(H):
                    yb[i, pl.ds(hh * 16, 16)] = ws * rb[i, pl.ds(hh * 16, 16)]
                return 0

            lax.fori_loop(0, CG, edge, 0)
            pltpu.sync_copy(yb, agg_sh.at[didx], add=True)

    plsc.subcore_barrier()
    _copy_out_1000(agg_sh, agg_hbm, cid, sid)


@functools.partial(
    pl.kernel, mesh=_SC_MESH,
    out_type=jax.ShapeDtypeStruct((NC, N_TIS, D), F32),
    scratch_types=[
        pltpu.VMEM((CG,), jnp.int32), pltpu.VMEM((CG,), jnp.int32),
        pltpu.VMEM((N_TIS,), F32),
        pltpu.VMEM((CG, D), F32), pltpu.VMEM((CG, D), F32),
        pltpu.VMEM((CG,), F32),
        pltpu.VMEM((63, D), F32),
        pltpu.VMEM_SHARED((N_TIS, D), F32),
    ])
def sc_gcn(*args):
    _gcn_sc_body(*args)


def _cross_sc_body(twt_hbm, twv_hbm, cwu_hbm, cl_hbm,
                   g1_hbm, g2_hbm, agg_hbm,
                   cidx, b1, ub, zb, agg_sh):
    cid = lax.axis_index("c")
    sid = lax.axis_index("s")
    wid = sid * NC + cid

    _zero_rows(zb, 63, D)
    _zero_spmem_1000(zb, agg_sh, sid)
    plsc.subcore_barrier()

    nchunk = N_CELL // CG
    for jj in range((nchunk + NW - 1) // NW):
        j = wid + NW * jj

        @pl.when(j < nchunk)
        def _():
            base = j * CG
            pltpu.sync_copy(cl_hbm.at[pl.ds(base, CG)], cidx)
            pltpu.sync_copy(twt_hbm.at[cidx], b1)
            pltpu.sync_copy(b1, g1_hbm.at[pl.ds(base, CG)])
            pltpu.sync_copy(twv_hbm.at[cidx], b1)
            pltpu.sync_copy(b1, g2_hbm.at[pl.ds(base, CG)])
            pltpu.sync_copy(cwu_hbm.at[pl.ds(base, CG)], ub)
            pltpu.sync_copy(ub, agg_sh.at[cidx], add=True)

    plsc.subcore_barrier()
    _copy_out_1000(agg_sh, agg_hbm, cid, sid)


@functools.partial(
    pl.kernel, mesh=_SC_MESH,
    out_type=[jax.ShapeDtypeStruct((N_CELL, D), F32),
              jax.ShapeDtypeStruct((N_CELL, D), F32),
              jax.ShapeDtypeStruct((NC, N_TIS, D), F32)],
    scratch_types=[
        pltpu.VMEM((CG,), jnp.int32),
        pltpu.VMEM((CG, D), F32), pltpu.VMEM((CG, D), F32),
        pltpu.VMEM((63, D), F32),
        pltpu.VMEM_SHARED((N_TIS, D), F32),
    ])
def sc_cross(*args):
    _cross_sc_body(*args)


def kernel(cell_x, cell_edge_index, tissue_x, tissue_edge_index,
           cluster_assignments, params):
    p = params
    csrc, cdst = cell_edge_index[0], cell_edge_index[1]
    tsrc, tdst = tissue_edge_index[0], tissue_edge_index[1]

    x = tc_dense(cell_x, p["in_proj"]["W"], p["in_proj"]["b"].reshape(1, D),
                 act="relu")
    for i in range(3):
        lp = p["cell_layers"][i]
        h, esd, pmax8 = tc_pre_gat(x, lp["W"], lp["att_src"].reshape(1, D),
                                   lp["att_dst"].reshape(1, D))
        pm = pmax8[0]
        gmaxv = jnp.concatenate([pm[:H] + pm[H:],
                                 jnp.full((H,), 1e30, F32)])
        out2, s2 = sc_gat(esd, h, csrc, cdst, gmaxv)
        ln = p["cell_lns"][i]
        x = tc_post_gat(out2, s2, lp["b"].reshape(1, D), x,
                        ln["g"].reshape(1, D), ln["b"].reshape(1, D))

    cell_feat, cwc, cwu = tc_precross(x, p["out_proj"]["W"],
                                      p["out_proj"]["b"].reshape(1, D),
                                      p["cross"]["Wc"], p["cross"]["Wu"])

    dinv8 = tc_count(tdst.reshape(E_TIS, 1), N_TIS, mode="dinv")
    dinv = dinv8[0]
    rcnt8 = tc_count(cluster_assignments.reshape(N_CELL, 1), N_TIS, mode="rcnt")

    t = tissue_x
    for i in range(2):
        cv = p["tissue_convs"][i]
        xw = tc_dense(t, cv["W"], block=N_TIS)
        a2 = sc_gcn(xw, dinv, tsrc, tdst)
        ln = p["tissue_lns"][i]
        t = tc_post_gcn(a2, xw, dinv.reshape(N_TIS, 1), cv["b"].reshape(1, D),
                        ln["g"].reshape(1, D), ln["b"].reshape(1, D), t)

    twt = tc_dense(t, p["cross"]["Wt"], block=N_TIS)
    twv = tc_dense(t, p["cross"]["Wv"], block=N_TIS)
    g1, g2, aggc = sc_cross(twt, twv, cwu, cluster_assignments)

    cell_attn = tc_cell_attn(cell_feat, cwc, g1, g2)
    lo, co = tc_heads(cell_attn, t, aggc, rcnt8[0].reshape(N_TIS, 1),
                      p["pool_cell"], p["pool_tissue"], p["fusion"],
                      p["clf"], p["count"])
    return lo[0:1], co[0:1]


# speedup vs baseline: 51.7218x; 51.7218x over previous
import functools

import jax
import jax.numpy as jnp
from jax import lax
from jax.experimental import pallas as pl
from jax.experimental.pallas import tpu as pltpu
from jax.experimental.pallas import tpu_sc as plsc

N_CELL, E_CELL, N_TIS, E_TIS, D, H, DH = 10000, 320000, 1000, 8000, 128, 8, 16
NC, NS = 2, 16
NW = NC * NS
CG = 80
CGG = 16
F32 = jnp.float32


def _dense_body(act, x_ref, w_ref, b_ref, o_ref):
    y = jnp.dot(x_ref[...], w_ref[...], preferred_element_type=F32)
    if b_ref is not None:
        y = y + b_ref[...]
    if act == "relu":
        y = jnp.maximum(y, 0.0)
    o_ref[...] = y


def tc_dense(x, w, b=None, act=None, block=2000):
    n, _ = x.shape
    k = w.shape[1]
    grid = max(n // block, 1)
    blk = n // grid
    if b is None:
        body = lambda x_ref, w_ref, o_ref: _dense_body(act, x_ref, w_ref, None, o_ref)
        in_specs = [pl.BlockSpec((blk, x.shape[1]), lambda i: (i, 0)),
                    pl.BlockSpec(w.shape, lambda i: (0, 0))]
        args = (x, w)
    else:
        body = functools.partial(_dense_body, act)
        in_specs = [pl.BlockSpec((blk, x.shape[1]), lambda i: (i, 0)),
                    pl.BlockSpec(w.shape, lambda i: (0, 0)),
                    pl.BlockSpec(b.shape, lambda i: (0, 0))]
        args = (x, w, b)
    return pl.pallas_call(
        body, grid=(grid,),
        in_specs=in_specs,
        out_specs=pl.BlockSpec((blk, k), lambda i: (i, 0)),
        out_shape=jax.ShapeDtypeStruct((n, k), F32),
    )(*args)


def _pre_gat_body(x_ref, w_ref, asrc_ref, adst_ref, h_ref, esd_ref, pmax_ref):
    h = jnp.dot(x_ref[...], w_ref[...], preferred_element_type=F32)
    h_ref[...] = h
    r = lax.broadcasted_iota(jnp.int32, (D, H), 0) // DH
    c = lax.broadcasted_iota(jnp.int32, (D, H), 1)
    pb = jnp.where(r == c, 1.0, 0.0).astype(F32)
    es = jnp.dot(h * asrc_ref[...], pb, preferred_element_type=F32)
    ed = jnp.dot(h * adst_ref[...], pb, preferred_element_type=F32)
    esd = jnp.concatenate([es, ed], axis=1)
    esd_ref[...] = jnp.concatenate(
        [esd, jnp.zeros((esd.shape[0], D - 2 * H), F32)], axis=1)
    pm = jnp.broadcast_to(jnp.max(esd, axis=0, keepdims=True), (8, 2 * H))

    @pl.when(pl.program_id(0) == 0)
    def _():
        pmax_ref[...] = pm

    @pl.when(pl.program_id(0) != 0)
    def _():
        pmax_ref[...] = jnp.maximum(pmax_ref[...], pm)


def tc_pre_gat(x, w, asrc, adst):
    n = x.shape[0]
    grid, blk = 5, n // 5
    return pl.pallas_call(
        _pre_gat_body, grid=(grid,),
        in_specs=[pl.BlockSpec((blk, D), lambda i: (i, 0)),
                  pl.BlockSpec((D, D), lambda i: (0, 0)),
                  pl.BlockSpec((1, D), lambda i: (0, 0)),
                  pl.BlockSpec((1, D), lambda i: (0, 0))],
        out_specs=[pl.BlockSpec((blk, D), lambda i: (i, 0)),
                   pl.BlockSpec((blk, D), lambda i: (i, 0)),
                   pl.BlockSpec((8, 2 * H), lambda i: (0, 0))],
        out_shape=[jax.ShapeDtypeStruct((n, D), F32),
                   jax.ShapeDtypeStruct((n, D), F32),
                   jax.ShapeDtypeStruct((8, 2 * H), F32)],
    )(x, w, asrc, adst)


def _post_gat_body(o2_ref, s2_ref, b_ref, res_ref, g_ref, bln_ref, o_ref):
    o = o2_ref[0] + o2_ref[1]
    s8 = (s2_ref[0] + s2_ref[1])[:, :H]
    r = lax.broadcasted_iota(jnp.int32, (H, D), 0)
    c = lax.broadcasted_iota(jnp.int32, (H, D), 1) // DH
    krep = jnp.where(r == c, 1.0, 0.0).astype(F32)
    srep = jnp.dot(s8, krep, preferred_element_type=F32)
    val = o / (srep + 1e-16) + b_ref[...] + res_ref[...]
    mu = jnp.mean(val, axis=1, keepdims=True)
    dv = val - mu
    var = jnp.mean(dv * dv, axis=1, keepdims=True)
    ln = dv / jnp.sqrt(var + 1e-5) * g_ref[...] + bln_ref[...]
    o_ref[...] = jnp.maximum(ln, 0.0)


def tc_post_gat(o2, s2, b, res, g, bln):
    n = res.shape[0]
    grid, blk = 5, n // 5
    return pl.pallas_call(
        _post_gat_body, grid=(grid,),
        in_specs=[pl.BlockSpec((2, blk, D), lambda i: (0, i, 0)),
                  pl.BlockSpec((2, blk, 2 * H), lambda i: (0, i, 0)),
                  pl.BlockSpec((1, D), lambda i: (0, 0)),
                  pl.BlockSpec((blk, D), lambda i: (i, 0)),
                  pl.BlockSpec((1, D), lambda i: (0, 0)),
                  pl.BlockSpec((1, D), lambda i: (0, 0))],
        out_specs=pl.BlockSpec((blk, D), lambda i: (i, 0)),
        out_shape=jax.ShapeDtypeStruct((n, D), F32),
    )(o2, s2, b, res, g, bln)


def _count_body(mode, idx_ref, out_ref):
    idx = idx_ref[...]
    cols = (lax.broadcasted_iota(jnp.int32, (idx.shape[0], 128), 1)
            + pl.program_id(0) * 128)
    m = jnp.where(idx == cols, 1.0, 0.0).astype(F32)
    deg = jnp.broadcast_to(jnp.sum(m, axis=0, keepdims=True), (8, 128))
    if mode == "dinv":
        out_ref[...] = lax.rsqrt(deg + 1.0)
    else:
        out_ref[...] = 1.0 / jnp.maximum(deg, 1.0)


def tc_count(idx_col, nseg, mode):
    n = idx_col.shape[0]
    out = pl.pallas_call(
        functools.partial(_count_body, mode), grid=(8,),
        in_specs=[pl.BlockSpec((n, 1), lambda i: (0, 0))],
        out_specs=pl.BlockSpec((8, 128), lambda i: (0, i)),
        out_shape=jax.ShapeDtypeStruct((8, 1024), F32),
    )(idx_col)
    return out[:, :nseg]


def _post_gcn_body(a2_ref, xw_ref, dv_ref, b_ref, g_ref, bln_ref, res_ref, o_ref):
    xw = xw_ref[...]
    dv = dv_ref[...]
    t2 = a2_ref[...] + dv * dv * xw + b_ref[...]
    mu = jnp.mean(t2, axis=1, keepdims=True)
    d_ = t2 - mu
    var = jnp.mean(d_ * d_, axis=1, keepdims=True)
    ln = d_ / jnp.sqrt(var + 1e-5) * g_ref[...] + bln_ref[...]
    o_ref[...] = jnp.maximum(ln, 0.0) + res_ref[...]


def tc_post_gcn(a2, xw, dinv_col, b, g, bln, res):
    n = xw.shape[0]
    return pl.pallas_call(
        _post_gcn_body,
        out_shape=jax.ShapeDtypeStruct((n, D), F32),
    )(a2, xw, dinv_col, b, g, bln, res)


def _precross_body(x_ref, wo_ref, bo_ref, wc_ref, wu_ref, cf_ref, cwc_ref, cwu_ref):
    cf = jnp.dot(x_ref[...], wo_ref[...], preferred_element_type=F32) + bo_ref[...]
    cf_ref[...] = cf
    cwc_ref[...] = jnp.dot(cf, wc_ref[...], preferred_element_type=F32)
    cwu_ref[...] = jnp.dot(cf, wu_ref[...], preferred_element_type=F32)


def tc_precross(x, wo, bo, wc, wu):
    n = x.shape[0]
    grid, blk = 5, n // 5
    wspec = pl.BlockSpec((D, D), lambda i: (0, 0))
    return pl.pallas_call(
        _precross_body, grid=(grid,),
        in_specs=[pl.BlockSpec((blk, D), lambda i: (i, 0)), wspec,
                  pl.BlockSpec((1, D), lambda i: (0, 0)), wspec, wspec],
        out_specs=[pl.BlockSpec((blk, D), lambda i: (i, 0))] * 3,
        out_shape=[jax.ShapeDtypeStruct((n, D), F32)] * 3,
    )(x, wo, bo, wc, wu)


def _cell_attn_body(cf_ref, cwc_ref, g1_ref, g2_ref, o_ref):
    score = jnp.sum(cwc_ref[...] * g1_ref[...], axis=1, keepdims=True) * (
        1.0 / jnp.sqrt(float(D)))
    alpha = 1.0 / (1.0 + jnp.exp(-score))
    o_ref[...] = cf_ref[...] + alpha * g2_ref[...]


def tc_cell_attn(cf, cwc, g1, g2):
    n = cf.shape[0]
    grid, blk = 5, n // 5
    spec = pl.BlockSpec((blk, D), lambda i: (i, 0))
    return pl.pallas_call(
        _cell_attn_body, grid=(grid,),
        in_specs=[spec] * 4,
        out_specs=spec,
        out_shape=jax.ShapeDtypeStruct((n, D), F32),
    )(cf, cwc, g1, g2)


def _heads_body(ca_ref, t_ref, agg_ref, rc_ref,
                pcw1_ref, pcb1_ref, pcw2_ref,
                ptw1_ref, ptb1_ref, ptw2_ref,
                fw1_ref, fb1_ref, fw2_ref, fb2_ref,
                cw1_ref, cb1_ref, cw2_ref, cb2_ref,
                kw1_ref, kb1_ref, kw2_ref, kb2_ref,
                lo_ref, co_ref):
    tis = t_ref[...] + agg_ref[...] * rc_ref[...]

    def pool(x, w1, b1, w2):
        s = jnp.dot(jnp.tanh(jnp.dot(x, w1, preferred_element_type=F32) + b1),
                    w2, preferred_element_type=F32)
        s = s - jnp.max(s, axis=0, keepdims=True)
        a = jnp.exp(s)
        a = a / jnp.sum(a, axis=0, keepdims=True)
        return jnp.sum(a * x, axis=0, keepdims=True)

    cr = pool(ca_ref[...], pcw1_ref[...], pcb1_ref[...], pcw2_ref[...])
    tr = pool(tis, ptw1_ref[...], ptb1_ref[...], ptw2_ref[...])
    fused = jnp.broadcast_to(jnp.concatenate([cr, tr], axis=1), (8, 2 * D))
    f1 = jnp.maximum(jnp.dot(fused, fw1_ref[...], preferred_element_type=F32)
                     + fb1_ref[...], 0.0)
    f2 = jnp.maximum(jnp.dot(f1, fw2_ref[...], preferred_element_type=F32)
                     + fb2_ref[...], 0.0)
    c1 = jnp.maximum(jnp.dot(f2, cw1_ref[...], preferred_element_type=F32)
                     + cb1_ref[...], 0.0)
    lo_ref[...] = jnp.dot(c1, cw2_ref[...], preferred_element_type=F32) + cb2_ref[...]
    k1 = jnp.maximum(jnp.dot(f2, kw1_ref[...], preferred_element_type=F32)
                     + kb1_ref[...], 0.0)
    co_ref[...] = jnp.dot(k1, kw2_ref[...], preferred_element_type=F32) + kb2_ref[...]


def tc_heads(ca, t, agg, rc_col, pc, pt, fu, cl, ct):
    return pl.pallas_call(
        _heads_body,
        out_shape=[jax.ShapeDtypeStruct((8, 4), F32),
                   jax.ShapeDtypeStruct((8, 10), F32)],
    )(ca, t, agg, rc_col,
      pc["W1"], pc["b1"].reshape(1, D), pc["w2"].reshape(D, 1),
      pt["W1"], pt["b1"].reshape(1, D), pt["w2"].reshape(D, 1),
      fu["W1"], fu["b1"].reshape(1, D), fu["W2"], fu["b2"].reshape(1, D // 2),
      cl["W1"], cl["b1"].reshape(1, D // 4), cl["W2"], cl["b2"].reshape(1, 4),
      ct["W1"], ct["b1"].reshape(1, D // 2), ct["W2"], ct["b2"].reshape(1, 10))


_GDN = lax.GatherDimensionNumbers(
    offset_dims=(), collapsed_slice_dims=(0,), start_index_map=(0,))


def _take16(x, idx):
    return lax.gather(x, idx[:, None], _GDN, (1,),
                      mode=lax.GatherScatterMode.PROMISE_IN_BOUNDS)


def _zero_rows(ref, rows, lanes):
    z = jnp.zeros((16,), F32)
    n16 = lanes // 16

    def body(i, _):
        r = i // n16
        c = (i - r * n16) * 16
        ref[r, pl.ds(c, 16)] = z
        return 0

    lax.fori_loop(0, rows * n16, body, 0)


@functools.lru_cache(maxsize=None)
def _sc_mesh():
    return plsc.VectorSubcoreMesh(core_axis_name="c", subcore_axis_name="s",
                                  num_cores=NC, num_subcores=NS)


_SC_PARAMS = pltpu.CompilerParams(needs_layout_passes=False)


def _route_half(dv16, base, half, dump):
    u = dv16 - base
    inh = (jnp.minimum(jnp.maximum(u + 1, 0), 1) *
           jnp.minimum(jnp.maximum(half - u, 0), 1))
    return u * inh + dump * (1 - inh)


def _gat_sc_body(esd_hbm, h_hbm, src_hbm, dst_hbm, gmax_hbm,
                 out_hbm, sacc_hbm,
                 sidx0, sidx1, didx0, didx1, dloc0, dloc1, dl80, dl81,
                 sb0, sb1, tb0, tb1, rb0, rb1, wb0, wb1, yb0, yb1,
                 gbuf, zb, sem_s0, sem_s1, sem_t0, sem_t1, sem_r0, sem_r1,
                 sem_y0, sem_y1, sem_w0, sem_w1,
                 out_sh, s_sh):
    cid = lax.axis_index("c")
    sid = lax.axis_index("s")
    SIDX = (sidx0, sidx1)
    DIDX = (didx0, didx1)
    DLOC = (dloc0, dloc1)
    DL8 = (dl80, dl81)
    SB = (sb0, sb1)
    TB = (tb0, tb1)
    RB = (rb0, rb1)
    WB = (wb0, wb1)
    YB = (yb0, yb1)
    SEM_S = (sem_s0, sem_s1)
    SEM_T = (sem_t0, sem_t1)
    SEM_R = (sem_r0, sem_r1)
    SEM_Y = (sem_y0, sem_y1)
    SEM_W = (sem_w0, sem_w1)

    _zero_rows(zb, 16, D)
    extra = jnp.minimum(jnp.maximum(sid - 14, 0), 1)

    def zchunk(k, _):
        pltpu.sync_copy(zb.at[pl.ds(0, 8)],
                        out_sh.at[pl.ds(sid * 624 + k * 8, 8)])
        return 0

    lax.fori_loop(0, 78 + 3 * extra, zchunk, 0)

    def zschunk(k, _):
        pltpu.sync_copy(zb.at[pl.ds(0, 8)],
                        s_sh.at[pl.ds(sid * 80 + k * 8, 8)])
        return 0

    lax.fori_loop(0, 10 - 3 * extra, zschunk, 0)
    plsc.subcore_barrier()

    pltpu.sync_copy(gmax_hbm, gbuf)
    gvec = gbuf[...]
    rot = jnp.remainder(lax.iota(jnp.int32, 16) + 8, 16)
    hsel = [jnp.full((16,), hh, jnp.int32) for hh in range(H)]
    lsel = [jnp.full((16,), ll, jnp.int32) for ll in range(16)]

    NCH = E_CELL // NC // NS // CGG
    ebase = (cid * NS + sid) * (E_CELL // NC // NS)

    def fetch(b, j):
        cb = ebase + j * CGG
        pltpu.sync_copy(src_hbm.at[pl.ds(cb, CGG)], SIDX[b])
        pltpu.sync_copy(dst_hbm.at[pl.ds(cb, CGG)], DIDX[b])
        pltpu.async_copy(esd_hbm.at[SIDX[b]], SB[b], SEM_S[b])
        pltpu.async_copy(esd_hbm.at[DIDX[b]], TB[b], SEM_T[b])
        pltpu.async_copy(h_hbm.at[SIDX[b]], RB[b], SEM_R[b])

    def wait_fetch(b):
        pltpu.make_async_copy(esd_hbm.at[SIDX[b]], SB[b], SEM_S[b]).wait()
        pltpu.make_async_copy(esd_hbm.at[DIDX[b]], TB[b], SEM_T[b]).wait()
        pltpu.make_async_copy(h_hbm.at[SIDX[b]], RB[b], SEM_R[b]).wait()

    def drain_scatter(b):
        pltpu.make_async_copy(YB[b], out_sh.at[DLOC[b]], SEM_Y[b]).wait()
        pltpu.make_async_copy(WB[b], s_sh.at[DL8[b]], SEM_W[b]).wait()

    def compute(b):
        sb, tb, rb, wb, yb = SB[b], TB[b], RB[b], WB[b], YB[b]
        dloc, dl8, didx = DLOC[b], DL8[b], DIDX[b]

        def group(g, _):
            loc16 = didx[pl.ds(g * 16, 16)]
            dloc[pl.ds(g * 16, 16)] = loc16
            dl8[pl.ds(g * 16, 16)] = lax.shift_right_logical(loc16, 3)
            for lane in range(16):
                i = g * 16 + lane
                sv = sb[i, pl.ds(0, 16)]
                tv = tb[i, pl.ds(0, 16)]
                e = sv + _take16(tv, rot)
                e = jnp.maximum(e, 0.2 * e)
                w = jnp.exp(e - gvec)
                lspl = _take16(loc16, lsel[lane])
                dmod = lspl - lax.shift_right_logical(lspl, 3) * 8
                for k in range(8):
                    mk = (1 - jnp.minimum(jnp.abs(dmod - k), 1)).astype(F32)
                    wb[i, pl.ds(k * 16, 16)] = mk * w
                for hh in range(H):
                    wbc = _take16(w, hsel[hh])
                    yb[i, pl.ds(hh * 16, 16)] = wbc * rb[i, pl.ds(hh * 16, 16)]
            return 0

        lax.fori_loop(0, CGG // 16, group, 0)
        pltpu.async_copy(yb, out_sh.at[dloc], SEM_Y[b], add=True)
        pltpu.async_copy(wb, s_sh.at[dl8], SEM_W[b], add=True)

    fetch(0, 0)

    def pair2(j2, _):
        for b in range(2):
            j = j2 * 2 + b

            @pl.when(j < NCH)
            def _():
                @pl.when(j + 1 < NCH)
                def _():
                    fetch(1 - b, j + 1)

                @pl.when(j >= 2)
                def _():
                    drain_scatter(b)

                wait_fetch(b)
                compute(b)
        return 0

    lax.fori_loop(0, (NCH + 1) // 2, pair2, 0)
    drain_scatter((NCH - 1) % 2)
    drain_scatter(NCH % 2)
    plsc.subcore_barrier()

    def ochunk(k, _):
        pltpu.sync_copy(out_sh.at[pl.ds(sid * 624 + k * 8, 8)],
                        out_hbm.at[cid, pl.ds(sid * 624 + k * 8, 8)])
        return 0

    lax.fori_loop(0, 78 + 2 * extra, ochunk, 0)

    def oschunk(k, _):
        pltpu.sync_copy(s_sh.at[pl.ds(sid * 80 + k * 8, 8)],
                        sacc_hbm.at[cid, pl.ds(sid * 80 + k * 8, 8)])
        return 0

    lax.fori_loop(0, 10 - 3 * extra, oschunk, 0)


@functools.lru_cache(maxsize=None)
def _sc_gat_fn():
    idx = pltpu.VMEM((CGG,), jnp.int32)
    buf = pltpu.VMEM((CGG, D), F32)
    sem = pltpu.SemaphoreType.DMA
    return pl.kernel(
        _gat_sc_body, mesh=_sc_mesh(),
        compiler_params=_SC_PARAMS,
        out_type=[jax.ShapeDtypeStruct((NC, N_CELL, D), F32),
                  jax.ShapeDtypeStruct((NC, 1256, D), F32)],
        scratch_types=(
            [idx] * 8 + [buf] * 10 +
            [pltpu.VMEM((16,), F32), pltpu.VMEM((16, D), F32)] +
            [sem] * 10 +
            [pltpu.VMEM_SHARED((N_CELL + 8, D), F32),
             pltpu.VMEM_SHARED((1256, D), F32)]
        ))


def sc_gat(*args):
    return _sc_gat_fn()(*args)


def _gcn_sc_body(xw_hbm, dinv_hbm, src_hbm, dst_hbm, agg_hbm,
                 sidx, didx, dloc, rb, yb, dvtbl, zb, agg_sh):
    cid = lax.axis_index("c")
    sid = lax.axis_index("s")
    half = N_TIS // NC
    base = cid * half
    DUMP = half

    _zero_rows(zb, 16, D)
    extra = jnp.minimum(jnp.maximum(sid - 14, 0), 1)

    def zchunk(k, _):
        pltpu.sync_copy(zb.at[pl.ds(0, 8)],
                        agg_sh.at[pl.ds(sid * 32 + k * 8, 8)])
        return 0

    lax.fori_loop(0, 4 - extra, zchunk, 0)
    pltpu.sync_copy(dinv_hbm, dvtbl)
    plsc.subcore_barrier()

    lsel = [jnp.full((16,), ll, jnp.int32) for ll in range(16)]

    nchunk = E_TIS // CG
    for jj in range((nchunk + NS - 1) // NS):
        j = sid + NS * jj

        @pl.when(j < nchunk)
        def _():
            cb = j * CG
            pltpu.sync_copy(src_hbm.at[pl.ds(cb, CG)], sidx)
            pltpu.sync_copy(dst_hbm.at[pl.ds(cb, CG)], didx)
            pltpu.sync_copy(xw_hbm.at[sidx], rb)

            def group(g, _):
                sv16 = sidx[pl.ds(g * 16, 16)]
                dv16 = didx[pl.ds(g * 16, 16)]
                wv = (plsc.load_gather(dvtbl, [sv16]) *
                      plsc.load_gather(dvtbl, [dv16]))
                loc16 = _route_half(dv16, base, half, DUMP)
                dloc[pl.ds(g * 16, 16)] = loc16
                for lane in range(16):
                    i = g * 16 + lane
                    wspl = _take16(wv, lsel[lane])
                    for hh in range(H):
                        yb[i, pl.ds(hh * 16, 16)] = (
                            wspl * rb[i, pl.ds(hh * 16, 16)])
                return 0

            lax.fori_loop(0, CG // 16, group, 0)
            pltpu.sync_copy(yb, agg_sh.at[dloc], add=True)

    plsc.subcore_barrier()

    def ochunk(k, _):
        pltpu.sync_copy(agg_sh.at[pl.ds(sid * 32 + k * 8, 8)],
                        agg_hbm.at[cid, pl.ds(sid * 32 + k * 8, 8)])
        return 0

    lax.fori_loop(0, 4 - extra, ochunk, 0)


@functools.lru_cache(maxsize=None)
def _sc_gcn_fn():
    return pl.kernel(
        _gcn_sc_body, mesh=_sc_mesh(),
        compiler_params=_SC_PARAMS,
        out_type=jax.ShapeDtypeStruct((NC, 504, D), F32),
        scratch_types=[
            pltpu.VMEM((CG,), jnp.int32), pltpu.VMEM((CG,), jnp.int32),
            pltpu.VMEM((CG,), jnp.int32),
            pltpu.VMEM((CG, D), F32), pltpu.VMEM((CG, D), F32),
            pltpu.VMEM((N_TIS,), F32), pltpu.VMEM((16, D), F32),
            pltpu.VMEM_SHARED((504, D), F32),
        ])


def sc_gcn(*args):
    return _sc_gcn_fn()(*args)


def _cross_sc_body(twt_hbm, twv_hbm, cwu_hbm, cl_hbm,
                   g1_hbm, g2_hbm, agg_hbm,
                   cidx, b1, dloc, ub, zb, agg_sh):
    cid = lax.axis_index("c")
    sid = lax.axis_index("s")
    wid = sid * NC + cid
    half = N_TIS // NC
    base = cid * half
    DUMP = half

    _zero_rows(zb, 16, D)
    extra = jnp.minimum(jnp.maximum(sid - 14, 0), 1)

    def zchunk(k, _):
        pltpu.sync_copy(zb.at[pl.ds(0, 8)],
                        agg_sh.at[pl.ds(sid * 32 + k * 8, 8)])
        return 0

    lax.fori_loop(0, 4 - extra, zchunk, 0)
    plsc.subcore_barrier()

    nchunk = N_CELL // CG
    for jj in range((nchunk + NW - 1) // NW):
        j = wid + NW * jj

        @pl.when(j < nchunk)
        def _():
            cb = j * CG
            pltpu.sync_copy(cl_hbm.at[pl.ds(cb, CG)], cidx)
            pltpu.sync_copy(twt_hbm.at[cidx], b1)
            pltpu.sync_copy(b1, g1_hbm.at[pl.ds(cb, CG)])
            pltpu.sync_copy(twv_hbm.at[cidx], b1)
            pltpu.sync_copy(b1, g2_hbm.at[pl.ds(cb, CG)])

    for jj in range((nchunk + NS - 1) // NS):
        j = sid + NS * jj

        @pl.when(j < nchunk)
        def _():
            cb = j * CG
            pltpu.sync_copy(cl_hbm.at[pl.ds(cb, CG)], cidx)
            pltpu.sync_copy(cwu_hbm.at[pl.ds(cb, CG)], ub)

            def group(g, _):
                dv16 = cidx[pl.ds(g * 16, 16)]
                dloc[pl.ds(g * 16, 16)] = _route_half(dv16, base, half, DUMP)
                return 0

            lax.fori_loop(0, CG // 16, group, 0)
            pltpu.sync_copy(ub, agg_sh.at[dloc], add=True)

    plsc.subcore_barrier()

    def ochunk(k, _):
        pltpu.sync_copy(agg_sh.at[pl.ds(sid * 32 + k * 8, 8)],
                        agg_hbm.at[cid, pl.ds(sid * 32 + k * 8, 8)])
        return 0

    lax.fori_loop(0, 4 - extra, ochunk, 0)


@functools.lru_cache(maxsize=None)
def _sc_cross_fn():
    return pl.kernel(
        _cross_sc_body, mesh=_sc_mesh(),
        compiler_params=_SC_PARAMS,
        out_type=[jax.ShapeDtypeStruct((N_CELL, D), F32),
                  jax.ShapeDtypeStruct((N_CELL, D), F32),
                  jax.ShapeDtypeStruct((NC, 504, D), F32)],
        scratch_types=[
            pltpu.VMEM((CG,), jnp.int32),
            pltpu.VMEM((CG, D), F32),
            pltpu.VMEM((CG,), jnp.int32),
            pltpu.VMEM((CG, D), F32), pltpu.VMEM((16, D), F32),
            pltpu.VMEM_SHARED((504, D), F32),
        ])


def sc_cross(*args):
    return _sc_cross_fn()(*args)


def kernel(cell_x, cell_edge_index, tissue_x, tissue_edge_index,
           cluster_assignments, params):
    p = params
    csrc, cdst = cell_edge_index[0], cell_edge_index[1]
    tsrc, tdst = tissue_edge_index[0], tissue_edge_index[1]

    x = tc_dense(cell_x, p["in_proj"]["W"], p["in_proj"]["b"].reshape(1, D),
                 act="relu")
    for i in range(3):
        lp = p["cell_layers"][i]
        h, esd, pmax8 = tc_pre_gat(x, lp["W"], lp["att_src"].reshape(1, D),
                                   lp["att_dst"].reshape(1, D))
        pm = pmax8[0]
        gmaxv = jnp.concatenate([pm[:H] + pm[H:],
                                 jnp.full((H,), 1e30, F32)])
        out2, sacc = sc_gat(esd, h, csrc, cdst, gmaxv)
        s2 = sacc[:, :N_CELL // 8, :].reshape(NC, N_CELL, 2 * H)
        ln = p["cell_lns"][i]
        x = tc_post_gat(out2, s2, lp["b"].reshape(1, D), x,
                        ln["g"].reshape(1, D), ln["b"].reshape(1, D))

    cell_feat, cwc, cwu = tc_precross(x, p["out_proj"]["W"],
                                      p["out_proj"]["b"].reshape(1, D),
                                      p["cross"]["Wc"], p["cross"]["Wu"])

    dinv8 = tc_count(tdst.reshape(E_TIS, 1), N_TIS, mode="dinv")
    dinv = dinv8[0]
    rcnt8 = tc_count(cluster_assignments.reshape(N_CELL, 1), N_TIS, mode="rcnt")

    t = tissue_x
    for i in range(2):
        cv = p["tissue_convs"][i]
        xw = tc_dense(t, cv["W"], block=N_TIS)
        a4 = sc_gcn(xw, dinv, tsrc, tdst)
        agg = a4[:, :N_TIS // NC, :].reshape(N_TIS, D)
        ln = p["tissue_lns"][i]
        t = tc_post_gcn(agg, xw, dinv.reshape(N_TIS, 1), cv["b"].reshape(1, D),
                        ln["g"].reshape(1, D), ln["b"].reshape(1, D), t)

    twt = tc_dense(t, p["cross"]["Wt"], block=N_TIS)
    twv = tc_dense(t, p["cross"]["Wv"], block=N_TIS)
    g1, g2, a4c = sc_cross(twt, twv, cwu, cluster_assignments)
    aggc = a4c[:, :N_TIS // NC, :].reshape(N_TIS, D)

    cell_attn = tc_cell_attn(cell_feat, cwc, g1, g2)
    lo, co = tc_heads(cell_attn, t, aggc, rcnt8[0].reshape(N_TIS, 1),
                      p["pool_cell"], p["pool_tissue"], p["fusion"],
                      p["clf"], p["count"])
    return lo[0:1], co[0:1]


# speedup vs baseline: 51.9667x; 1.0047x over previous
import functools

import jax
import jax.numpy as jnp
from jax import lax
from jax.experimental import pallas as pl
from jax.experimental.pallas import tpu as pltpu
from jax.experimental.pallas import tpu_sc as plsc

N_CELL, E_CELL, N_TIS, E_TIS, D, H, DH = 10000, 320000, 1000, 8000, 128, 8, 16
NC, NS = 2, 16
NW = NC * NS
CG = 80
CGG = 16
F32 = jnp.float32


def _dense_body(act, x_ref, w_ref, b_ref, o_ref):
    y = jnp.dot(x_ref[...], w_ref[...], preferred_element_type=F32)
    if b_ref is not None:
        y = y + b_ref[...]
    if act == "relu":
        y = jnp.maximum(y, 0.0)
    o_ref[...] = y


def tc_dense(x, w, b=None, act=None, block=2000):
    n, _ = x.shape
    k = w.shape[1]
    grid = max(n // block, 1)
    blk = n // grid
    if b is None:
        body = lambda x_ref, w_ref, o_ref: _dense_body(act, x_ref, w_ref, None, o_ref)
        in_specs = [pl.BlockSpec((blk, x.shape[1]), lambda i: (i, 0)),
                    pl.BlockSpec(w.shape, lambda i: (0, 0))]
        args = (x, w)
    else:
        body = functools.partial(_dense_body, act)
        in_specs = [pl.BlockSpec((blk, x.shape[1]), lambda i: (i, 0)),
                    pl.BlockSpec(w.shape, lambda i: (0, 0)),
                    pl.BlockSpec(b.shape, lambda i: (0, 0))]
        args = (x, w, b)
    return pl.pallas_call(
        body, grid=(grid,),
        in_specs=in_specs,
        out_specs=pl.BlockSpec((blk, k), lambda i: (i, 0)),
        out_shape=jax.ShapeDtypeStruct((n, k), F32),
    )(*args)


def _pre_gat_body(x_ref, w_ref, asrc_ref, adst_ref, h_ref, esd_ref, pmax_ref):
    h = jnp.dot(x_ref[...], w_ref[...], preferred_element_type=F32)
    h_ref[...] = h
    r = lax.broadcasted_iota(jnp.int32, (D, H), 0) // DH
    c = lax.broadcasted_iota(jnp.int32, (D, H), 1)
    pb = jnp.where(r == c, 1.0, 0.0).astype(F32)
    es = jnp.dot(h * asrc_ref[...], pb, preferred_element_type=F32)
    ed = jnp.dot(h * adst_ref[...], pb, preferred_element_type=F32)
    esd = jnp.concatenate([es, ed], axis=1)
    esd_ref[...] = jnp.concatenate(
        [esd, jnp.zeros((esd.shape[0], D - 2 * H), F32)], axis=1)
    pm = jnp.broadcast_to(jnp.max(esd, axis=0, keepdims=True), (8, 2 * H))

    @pl.when(pl.program_id(0) == 0)
    def _():
        pmax_ref[...] = pm

    @pl.when(pl.program_id(0) != 0)
    def _():
        pmax_ref[...] = jnp.maximum(pmax_ref[...], pm)


def tc_pre_gat(x, w, asrc, adst):
    n = x.shape[0]
    grid, blk = 5, n // 5
    return pl.pallas_call(
        _pre_gat_body, grid=(grid,),
        in_specs=[pl.BlockSpec((blk, D), lambda i: (i, 0)),
                  pl.BlockSpec((D, D), lambda i: (0, 0)),
                  pl.BlockSpec((1, D), lambda i: (0, 0)),
                  pl.BlockSpec((1, D), lambda i: (0, 0))],
        out_specs=[pl.BlockSpec((blk, D), lambda i: (i, 0)),
                   pl.BlockSpec((blk, D), lambda i: (i, 0)),
                   pl.BlockSpec((8, 2 * H), lambda i: (0, 0))],
        out_shape=[jax.ShapeDtypeStruct((n, D), F32),
                   jax.ShapeDtypeStruct((n, D), F32),
                   jax.ShapeDtypeStruct((8, 2 * H), F32)],
    )(x, w, asrc, adst)


def _post_gat_body(o2_ref, s2_ref, b_ref, res_ref, g_ref, bln_ref, o_ref):
    o = o2_ref[0] + o2_ref[1]
    s8 = (s2_ref[0] + s2_ref[1])[:, :H]
    r = lax.broadcasted_iota(jnp.int32, (H, D), 0)
    c = lax.broadcasted_iota(jnp.int32, (H, D), 1) // DH
    krep = jnp.where(r == c, 1.0, 0.0).astype(F32)
    srep = jnp.dot(s8, krep, preferred_element_type=F32)
    val = o / (srep + 1e-16) + b_ref[...] + res_ref[...]
    mu = jnp.mean(val, axis=1, keepdims=True)
    dv = val - mu
    var = jnp.mean(dv * dv, axis=1, keepdims=True)
    ln = dv / jnp.sqrt(var + 1e-5) * g_ref[...] + bln_ref[...]
    o_ref[...] = jnp.maximum(ln, 0.0)


def tc_post_gat(o2, s2, b, res, g, bln):
    n = res.shape[0]
    grid, blk = 5, n // 5
    return pl.pallas_call(
        _post_gat_body, grid=(grid,),
        in_specs=[pl.BlockSpec((2, blk, D), lambda i: (0, i, 0)),
                  pl.BlockSpec((2, blk, 2 * H), lambda i: (0, i, 0)),
                  pl.BlockSpec((1, D), lambda i: (0, 0)),
                  pl.BlockSpec((blk, D), lambda i: (i, 0)),
                  pl.BlockSpec((1, D), lambda i: (0, 0)),
                  pl.BlockSpec((1, D), lambda i: (0, 0))],
        out_specs=pl.BlockSpec((blk, D), lambda i: (i, 0)),
        out_shape=jax.ShapeDtypeStruct((n, D), F32),
    )(o2, s2, b, res, g, bln)


def _count_body(mode, idx_ref, out_ref):
    idx = idx_ref[...]
    cols = (lax.broadcasted_iota(jnp.int32, (idx.shape[0], 128), 1)
            + pl.program_id(0) * 128)
    m = jnp.where(idx == cols, 1.0, 0.0).astype(F32)
    deg = jnp.broadcast_to(jnp.sum(m, axis=0, keepdims=True), (8, 128))
    if mode == "dinv":
        out_ref[...] = lax.rsqrt(deg + 1.0)
    else:
        out_ref[...] = 1.0 / jnp.maximum(deg, 1.0)


def tc_count(idx_col, nseg, mode):
    n = idx_col.shape[0]
    out = pl.pallas_call(
        functools.partial(_count_body, mode), grid=(8,),
        in_specs=[pl.BlockSpec((n, 1), lambda i: (0, 0))],
        out_specs=pl.BlockSpec((8, 128), lambda i: (0, i)),
        out_shape=jax.ShapeDtypeStruct((8, 1024), F32),
    )(idx_col)
    return out[:, :nseg]


def _post_gcn_body(a2_ref, xw_ref, dv_ref, b_ref, g_ref, bln_ref, res_ref, o_ref):
    xw = xw_ref[...]
    dv = dv_ref[...]
    t2 = a2_ref[...] + dv * dv * xw + b_ref[...]
    mu = jnp.mean(t2, axis=1, keepdims=True)
    d_ = t2 - mu
    var = jnp.mean(d_ * d_, axis=1, keepdims=True)
    ln = d_ / jnp.sqrt(var + 1e-5) * g_ref[...] + bln_ref[...]
    o_ref[...] = jnp.maximum(ln, 0.0) + res_ref[...]


def tc_post_gcn(a2, xw, dinv_col, b, g, bln, res):
    n = xw.shape[0]
    return pl.pallas_call(
        _post_gcn_body,
        out_shape=jax.ShapeDtypeStruct((n, D), F32),
    )(a2, xw, dinv_col, b, g, bln, res)


def _precross_body(x_ref, wo_ref, bo_ref, wc_ref, wu_ref, cf_ref, cwc_ref, cwu_ref):
    cf = jnp.dot(x_ref[...], wo_ref[...], preferred_element_type=F32) + bo_ref[...]
    cf_ref[...] = cf
    cwc_ref[...] = jnp.dot(cf, wc_ref[...], preferred_element_type=F32)
    cwu_ref[...] = jnp.dot(cf, wu_ref[...], preferred_element_type=F32)


def tc_precross(x, wo, bo, wc, wu):
    n = x.shape[0]
    grid, blk = 5, n // 5
    wspec = pl.BlockSpec((D, D), lambda i: (0, 0))
    return pl.pallas_call(
        _precross_body, grid=(grid,),
        in_specs=[pl.BlockSpec((blk, D), lambda i: (i, 0)), wspec,
                  pl.BlockSpec((1, D), lambda i: (0, 0)), wspec, wspec],
        out_specs=[pl.BlockSpec((blk, D), lambda i: (i, 0))] * 3,
        out_shape=[jax.ShapeDtypeStruct((n, D), F32)] * 3,
    )(x, wo, bo, wc, wu)


def _cell_attn_body(cf_ref, cwc_ref, g1_ref, g2_ref, o_ref):
    score = jnp.sum(cwc_ref[...] * g1_ref[...], axis=1, keepdims=True) * (
        1.0 / jnp.sqrt(float(D)))
    alpha = 1.0 / (1.0 + jnp.exp(-score))
    o_ref[...] = cf_ref[...] + alpha * g2_ref[...]


def tc_cell_attn(cf, cwc, g1, g2):
    n = cf.shape[0]
    grid, blk = 5, n // 5
    spec = pl.BlockSpec((blk, D), lambda i: (i, 0))
    return pl.pallas_call(
        _cell_attn_body, grid=(grid,),
        in_specs=[spec] * 4,
        out_specs=spec,
        out_shape=jax.ShapeDtypeStruct((n, D), F32),
    )(cf, cwc, g1, g2)


def _heads_body(ca_ref, t_ref, agg_ref, rc_ref,
                pcw1_ref, pcb1_ref, pcw2_ref,
                ptw1_ref, ptb1_ref, ptw2_ref,
                fw1_ref, fb1_ref, fw2_ref, fb2_ref,
                cw1_ref, cb1_ref, cw2_ref, cb2_ref,
                kw1_ref, kb1_ref, kw2_ref, kb2_ref,
                lo_ref, co_ref):
    tis = t_ref[...] + agg_ref[...] * rc_ref[...]

    def pool(x, w1, b1, w2):
        s = jnp.dot(jnp.tanh(jnp.dot(x, w1, preferred_element_type=F32) + b1),
                    w2, preferred_element_type=F32)
        s = s - jnp.max(s, axis=0, keepdims=True)
        a = jnp.exp(s)
        a = a / jnp.sum(a, axis=0, keepdims=True)
        return jnp.sum(a * x, axis=0, keepdims=True)

    cr = pool(ca_ref[...], pcw1_ref[...], pcb1_ref[...], pcw2_ref[...])
    tr = pool(tis, ptw1_ref[...], ptb1_ref[...], ptw2_ref[...])
    fused = jnp.broadcast_to(jnp.concatenate([cr, tr], axis=1), (8, 2 * D))
    f1 = jnp.maximum(jnp.dot(fused, fw1_ref[...], preferred_element_type=F32)
                     + fb1_ref[...], 0.0)
    f2 = jnp.maximum(jnp.dot(f1, fw2_ref[...], preferred_element_type=F32)
                     + fb2_ref[...], 0.0)
    c1 = jnp.maximum(jnp.dot(f2, cw1_ref[...], preferred_element_type=F32)
                     + cb1_ref[...], 0.0)
    lo_ref[...] = jnp.dot(c1, cw2_ref[...], preferred_element_type=F32) + cb2_ref[...]
    k1 = jnp.maximum(jnp.dot(f2, kw1_ref[...], preferred_element_type=F32)
                     + kb1_ref[...], 0.0)
    co_ref[...] = jnp.dot(k1, kw2_ref[...], preferred_element_type=F32) + kb2_ref[...]


def tc_heads(ca, t, agg, rc_col, pc, pt, fu, cl, ct):
    return pl.pallas_call(
        _heads_body,
        out_shape=[jax.ShapeDtypeStruct((8, 4), F32),
                   jax.ShapeDtypeStruct((8, 10), F32)],
    )(ca, t, agg, rc_col,
      pc["W1"], pc["b1"].reshape(1, D), pc["w2"].reshape(D, 1),
      pt["W1"], pt["b1"].reshape(1, D), pt["w2"].reshape(D, 1),
      fu["W1"], fu["b1"].reshape(1, D), fu["W2"], fu["b2"].reshape(1, D // 2),
      cl["W1"], cl["b1"].reshape(1, D // 4), cl["W2"], cl["b2"].reshape(1, 4),
      ct["W1"], ct["b1"].reshape(1, D // 2), ct["W2"], ct["b2"].reshape(1, 10))


_GDN = lax.GatherDimensionNumbers(
    offset_dims=(), collapsed_slice_dims=(0,), start_index_map=(0,))


def _take16(x, idx):
    return lax.gather(x, idx[:, None], _GDN, (1,),
                      mode=lax.GatherScatterMode.PROMISE_IN_BOUNDS)


def _zero_rows(ref, rows, lanes):
    z = jnp.zeros((16,), F32)
    n16 = lanes // 16

    def body(i, _):
        r = i // n16
        c = (i - r * n16) * 16
        ref[r, pl.ds(c, 16)] = z
        return 0

    lax.fori_loop(0, rows * n16, body, 0)


@functools.lru_cache(maxsize=None)
def _sc_mesh():
    return plsc.VectorSubcoreMesh(core_axis_name="c", subcore_axis_name="s",
                                  num_cores=NC, num_subcores=NS)


_SC_PARAMS = pltpu.CompilerParams(needs_layout_passes=False)


def _route_half(dv16, base, half, dump):
    u = dv16 - base
    inh = (jnp.minimum(jnp.maximum(u + 1, 0), 1) *
           jnp.minimum(jnp.maximum(half - u, 0), 1))
    return u * inh + dump * (1 - inh)


def _gat_sc_body(esd_hbm, h_hbm, src_hbm, dst_hbm, gmax_hbm,
                 out_hbm, sacc_hbm,
                 sidx0, sidx1, sidx2, didx0, didx1, didx2,
                 dloc0, dloc1, dloc2, dl80, dl81, dl82,
                 sb0, sb1, sb2, tb0, tb1, tb2, rb0, rb1, rb2,
                 wb0, wb1, wb2, yb0, yb1, yb2,
                 gbuf, zb,
                 sem_s0, sem_s1, sem_s2, sem_t0, sem_t1, sem_t2,
                 sem_r0, sem_r1, sem_r2, sem_y0, sem_y1, sem_y2,
                 sem_w0, sem_w1, sem_w2,
                 out_sh, s_sh):
    cid = lax.axis_index("c")
    sid = lax.axis_index("s")
    SIDX = (sidx0, sidx1, sidx2)
    DIDX = (didx0, didx1, didx2)
    DLOC = (dloc0, dloc1, dloc2)
    DL8 = (dl80, dl81, dl82)
    SB = (sb0, sb1, sb2)
    TB = (tb0, tb1, tb2)
    RB = (rb0, rb1, rb2)
    WB = (wb0, wb1, wb2)
    YB = (yb0, yb1, yb2)
    SEM_S = (sem_s0, sem_s1, sem_s2)
    SEM_T = (sem_t0, sem_t1, sem_t2)
    SEM_R = (sem_r0, sem_r1, sem_r2)
    SEM_Y = (sem_y0, sem_y1, sem_y2)
    SEM_W = (sem_w0, sem_w1, sem_w2)

    _zero_rows(zb, 16, D)
    extra = jnp.minimum(jnp.maximum(sid - 14, 0), 1)

    def zchunk(k, _):
        pltpu.sync_copy(zb.at[pl.ds(0, 8)],
                        out_sh.at[pl.ds(sid * 624 + k * 8, 8)])
        return 0

    lax.fori_loop(0, 78 + 3 * extra, zchunk, 0)

    def zschunk(k, _):
        pltpu.sync_copy(zb.at[pl.ds(0, 8)],
                        s_sh.at[pl.ds(sid * 80 + k * 8, 8)])
        return 0

    lax.fori_loop(0, 10 - 3 * extra, zschunk, 0)
    plsc.subcore_barrier()

    pltpu.sync_copy(gmax_hbm, gbuf)
    gvec = gbuf[...]
    rot = jnp.remainder(lax.iota(jnp.int32, 16) + 8, 16)
    hsel = [jnp.full((16,), hh, jnp.int32) for hh in range(H)]
    lsel = [jnp.full((16,), ll, jnp.int32) for ll in range(16)]

    NCH = E_CELL // NC // NS // CGG
    ebase = (cid * NS + sid) * (E_CELL // NC // NS)

    def fetch(b, j):
        cb = ebase + j * CGG
        pltpu.sync_copy(src_hbm.at[pl.ds(cb, CGG)], SIDX[b])
        pltpu.sync_copy(dst_hbm.at[pl.ds(cb, CGG)], DIDX[b])
        pltpu.async_copy(esd_hbm.at[SIDX[b]], SB[b], SEM_S[b])
        pltpu.async_copy(esd_hbm.at[DIDX[b]], TB[b], SEM_T[b])
        pltpu.async_copy(h_hbm.at[SIDX[b]], RB[b], SEM_R[b])

    def wait_fetch(b):
        pltpu.make_async_copy(esd_hbm.at[SIDX[b]], SB[b], SEM_S[b]).wait()
        pltpu.make_async_copy(esd_hbm.at[DIDX[b]], TB[b], SEM_T[b]).wait()
        pltpu.make_async_copy(h_hbm.at[SIDX[b]], RB[b], SEM_R[b]).wait()

    def drain_scatter(b):
        pltpu.make_async_copy(YB[b], out_sh.at[DLOC[b]], SEM_Y[b]).wait()
        pltpu.make_async_copy(WB[b], s_sh.at[DL8[b]], SEM_W[b]).wait()

    def compute(b):
        sb, tb, rb, wb, yb = SB[b], TB[b], RB[b], WB[b], YB[b]
        dloc, dl8, didx = DLOC[b], DL8[b], DIDX[b]

        def group(g, _):
            loc16 = didx[pl.ds(g * 16, 16)]
            dloc[pl.ds(g * 16, 16)] = loc16
            dl8[pl.ds(g * 16, 16)] = lax.shift_right_logical(loc16, 3)
            for lane in range(16):
                i = g * 16 + lane
                sv = sb[i, pl.ds(0, 16)]
                tv = tb[i, pl.ds(0, 16)]
                e = sv + _take16(tv, rot)
                e = jnp.maximum(e, 0.2 * e)
                w = jnp.exp(e - gvec)
                lspl = _take16(loc16, lsel[lane])
                dmod = lspl - lax.shift_right_logical(lspl, 3) * 8
                for k in range(8):
                    mk = (1 - jnp.minimum(jnp.abs(dmod - k), 1)).astype(F32)
                    wb[i, pl.ds(k * 16, 16)] = mk * w
                for hh in range(H):
                    wbc = _take16(w, hsel[hh])
                    yb[i, pl.ds(hh * 16, 16)] = wbc * rb[i, pl.ds(hh * 16, 16)]
            return 0

        lax.fori_loop(0, CGG // 16, group, 0)
        pltpu.async_copy(yb, out_sh.at[dloc], SEM_Y[b], add=True)
        pltpu.async_copy(wb, s_sh.at[dl8], SEM_W[b], add=True)

    fetch(0, 0)
    fetch(1, 1)

    def triple(j3, _):
        for b in range(3):
            j = j3 * 3 + b

            @pl.when(j < NCH)
            def _():
                @pl.when(j + 2 < NCH)
                def _():
                    fetch((b + 2) % 3, j + 2)

                @pl.when(j >= 3)
                def _():
                    drain_scatter(b)

                wait_fetch(b)
                compute(b)
        return 0

    lax.fori_loop(0, (NCH + 2) // 3, triple, 0)
    for b in range(3):
        drain_scatter(b)
    plsc.subcore_barrier()

    def ochunk(k, _):
        pltpu.sync_copy(out_sh.at[pl.ds(sid * 624 + k * 8, 8)],
                        out_hbm.at[cid, pl.ds(sid * 624 + k * 8, 8)])
        return 0

    lax.fori_loop(0, 78 + 2 * extra, ochunk, 0)

    def oschunk(k, _):
        pltpu.sync_copy(s_sh.at[pl.ds(sid * 80 + k * 8, 8)],
                        sacc_hbm.at[cid, pl.ds(sid * 80 + k * 8, 8)])
        return 0

    lax.fori_loop(0, 10 - 3 * extra, oschunk, 0)


@functools.lru_cache(maxsize=None)
def _sc_gat_fn():
    idx = pltpu.VMEM((CGG,), jnp.int32)
    buf = pltpu.VMEM((CGG, D), F32)
    sem = pltpu.SemaphoreType.DMA
    return pl.kernel(
        _gat_sc_body, mesh=_sc_mesh(),
        compiler_params=_SC_PARAMS,
        out_type=[jax.ShapeDtypeStruct((NC, N_CELL, D), F32),
                  jax.ShapeDtypeStruct((NC, 1256, D), F32)],
        scratch_types=(
            [idx] * 12 + [buf] * 15 +
            [pltpu.VMEM((16,), F32), pltpu.VMEM((16, D), F32)] +
            [sem] * 15 +
            [pltpu.VMEM_SHARED((N_CELL + 8, D), F32),
             pltpu.VMEM_SHARED((1256, D), F32)]
        ))


def sc_gat(*args):
    return _sc_gat_fn()(*args)


def _gcn_sc_body(xw_hbm, dinv_hbm, src_hbm, dst_hbm, agg_hbm,
                 sidx, didx, dloc, rb, yb, dvtbl, zb, agg_sh):
    cid = lax.axis_index("c")
    sid = lax.axis_index("s")
    half = N_TIS // NC
    base = cid * half
    DUMP = half

    _zero_rows(zb, 16, D)
    extra = jnp.minimum(jnp.maximum(sid - 14, 0), 1)

    def zchunk(k, _):
        pltpu.sync_copy(zb.at[pl.ds(0, 8)],
                        agg_sh.at[pl.ds(sid * 32 + k * 8, 8)])
        return 0

    lax.fori_loop(0, 4 - extra, zchunk, 0)
    pltpu.sync_copy(dinv_hbm, dvtbl)
    plsc.subcore_barrier()

    lsel = [jnp.full((16,), ll, jnp.int32) for ll in range(16)]

    nchunk = E_TIS // CG
    for jj in range((nchunk + NS - 1) // NS):
        j = sid + NS * jj

        @pl.when(j < nchunk)
        def _():
            cb = j * CG
            pltpu.sync_copy(src_hbm.at[pl.ds(cb, CG)], sidx)
            pltpu.sync_copy(dst_hbm.at[pl.ds(cb, CG)], didx)
            pltpu.sync_copy(xw_hbm.at[sidx], rb)

            def group(g, _):
                sv16 = sidx[pl.ds(g * 16, 16)]
                dv16 = didx[pl.ds(g * 16, 16)]
                wv = (plsc.load_gather(dvtbl, [sv16]) *
                      plsc.load_gather(dvtbl, [dv16]))
                loc16 = _route_half(dv16, base, half, DUMP)
                dloc[pl.ds(g * 16, 16)] = loc16
                for lane in range(16):
                    i = g * 16 + lane
                    wspl = _take16(wv, lsel[lane])
                    for hh in range(H):
                        yb[i, pl.ds(hh * 16, 16)] = (
                            wspl * rb[i, pl.ds(hh * 16, 16)])
                return 0

            lax.fori_loop(0, CG // 16, group, 0)
            pltpu.sync_copy(yb, agg_sh.at[dloc], add=True)

    plsc.subcore_barrier()

    def ochunk(k, _):
        pltpu.sync_copy(agg_sh.at[pl.ds(sid * 32 + k * 8, 8)],
                        agg_hbm.at[cid, pl.ds(sid * 32 + k * 8, 8)])
        return 0

    lax.fori_loop(0, 4 - extra, ochunk, 0)


@functools.lru_cache(maxsize=None)
def _sc_gcn_fn():
    return pl.kernel(
        _gcn_sc_body, mesh=_sc_mesh(),
        compiler_params=_SC_PARAMS,
        out_type=jax.ShapeDtypeStruct((NC, 504, D), F32),
        scratch_types=[
            pltpu.VMEM((CG,), jnp.int32), pltpu.VMEM((CG,), jnp.int32),
            pltpu.VMEM((CG,), jnp.int32),
            pltpu.VMEM((CG, D), F32), pltpu.VMEM((CG, D), F32),
            pltpu.VMEM((N_TIS,), F32), pltpu.VMEM((16, D), F32),
            pltpu.VMEM_SHARED((504, D), F32),
        ])


def sc_gcn(*args):
    return _sc_gcn_fn()(*args)


def _cross_sc_body(twt_hbm, twv_hbm, cwu_hbm, cl_hbm,
                   g1_hbm, g2_hbm, agg_hbm,
                   cidx, b1, dloc, ub, zb, agg_sh):
    cid = lax.axis_index("c")
    sid = lax.axis_index("s")
    wid = sid * NC + cid
    half = N_TIS // NC
    base = cid * half
    DUMP = half

    _zero_rows(zb, 16, D)
    extra = jnp.minimum(jnp.maximum(sid - 14, 0), 1)

    def zchunk(k, _):
        pltpu.sync_copy(zb.at[pl.ds(0, 8)],
                        agg_sh.at[pl.ds(sid * 32 + k * 8, 8)])
        return 0

    lax.fori_loop(0, 4 - extra, zchunk, 0)
    plsc.subcore_barrier()

    nchunk = N_CELL // CG
    for jj in range((nchunk + NW - 1) // NW):
        j = wid + NW * jj

        @pl.when(j < nchunk)
        def _():
            cb = j * CG
            pltpu.sync_copy(cl_hbm.at[pl.ds(cb, CG)], cidx)
            pltpu.sync_copy(twt_hbm.at[cidx], b1)
            pltpu.sync_copy(b1, g1_hbm.at[pl.ds(cb, CG)])
            pltpu.sync_copy(twv_hbm.at[cidx], b1)
            pltpu.sync_copy(b1, g2_hbm.at[pl.ds(cb, CG)])

    for jj in range((nchunk + NS - 1) // NS):
        j = sid + NS * jj

        @pl.when(j < nchunk)
        def _():
            cb = j * CG
            pltpu.sync_copy(cl_hbm.at[pl.ds(cb, CG)], cidx)
            pltpu.sync_copy(cwu_hbm.at[pl.ds(cb, CG)], ub)

            def group(g, _):
                dv16 = cidx[pl.ds(g * 16, 16)]
                dloc[pl.ds(g * 16, 16)] = _route_half(dv16, base, half, DUMP)
                return 0

            lax.fori_loop(0, CG // 16, group, 0)
            pltpu.sync_copy(ub, agg_sh.at[dloc], add=True)

    plsc.subcore_barrier()

    def ochunk(k, _):
        pltpu.sync_copy(agg_sh.at[pl.ds(sid * 32 + k * 8, 8)],
                        agg_hbm.at[cid, pl.ds(sid * 32 + k * 8, 8)])
        return 0

    lax.fori_loop(0, 4 - extra, ochunk, 0)


@functools.lru_cache(maxsize=None)
def _sc_cross_fn():
    return pl.kernel(
        _cross_sc_body, mesh=_sc_mesh(),
        compiler_params=_SC_PARAMS,
        out_type=[jax.ShapeDtypeStruct((N_CELL, D), F32),
                  jax.ShapeDtypeStruct((N_CELL, D), F32),
                  jax.ShapeDtypeStruct((NC, 504, D), F32)],
        scratch_types=[
            pltpu.VMEM((CG,), jnp.int32),
            pltpu.VMEM((CG, D), F32),
            pltpu.VMEM((CG,), jnp.int32),
            pltpu.VMEM((CG, D), F32), pltpu.VMEM((16, D), F32),
            pltpu.VMEM_SHARED((504, D), F32),
        ])


def sc_cross(*args):
    return _sc_cross_fn()(*args)


def kernel(cell_x, cell_edge_index, tissue_x, tissue_edge_index,
           cluster_assignments, params):
    p = params
    csrc, cdst = cell_edge_index[0], cell_edge_index[1]
    tsrc, tdst = tissue_edge_index[0], tissue_edge_index[1]

    x = tc_dense(cell_x, p["in_proj"]["W"], p["in_proj"]["b"].reshape(1, D),
                 act="relu")
    for i in range(3):
        lp = p["cell_layers"][i]
        h, esd, pmax8 = tc_pre_gat(x, lp["W"], lp["att_src"].reshape(1, D),
                                   lp["att_dst"].reshape(1, D))
        pm = pmax8[0]
        gmaxv = jnp.concatenate([pm[:H] + pm[H:],
                                 jnp.full((H,), 1e30, F32)])
        out2, sacc = sc_gat(esd, h, csrc, cdst, gmaxv)
        s2 = sacc[:, :N_CELL // 8, :].reshape(NC, N_CELL, 2 * H)
        ln = p["cell_lns"][i]
        x = tc_post_gat(out2, s2, lp["b"].reshape(1, D), x,
                        ln["g"].reshape(1, D), ln["b"].reshape(1, D))

    cell_feat, cwc, cwu = tc_precross(x, p["out_proj"]["W"],
                                      p["out_proj"]["b"].reshape(1, D),
                                      p["cross"]["Wc"], p["cross"]["Wu"])

    dinv8 = tc_count(tdst.reshape(E_TIS, 1), N_TIS, mode="dinv")
    dinv = dinv8[0]
    rcnt8 = tc_count(cluster_assignments.reshape(N_CELL, 1), N_TIS, mode="rcnt")

    t = tissue_x
    for i in range(2):
        cv = p["tissue_convs"][i]
        xw = tc_dense(t, cv["W"], block=N_TIS)
        a4 = sc_gcn(xw, dinv, tsrc, tdst)
        agg = a4[:, :N_TIS // NC, :].reshape(N_TIS, D)
        ln = p["tissue_lns"][i]
        t = tc_post_gcn(agg, xw, dinv.reshape(N_TIS, 1), cv["b"].reshape(1, D),
                        ln["g"].reshape(1, D), ln["b"].reshape(1, D), t)

    twt = tc_dense(t, p["cross"]["Wt"], block=N_TIS)
    twv = tc_dense(t, p["cross"]["Wv"], block=N_TIS)
    g1, g2, a4c = sc_cross(twt, twv, cwu, cluster_assignments)
    aggc = a4c[:, :N_TIS // NC, :].reshape(N_TIS, D)

    cell_attn = tc_cell_attn(cell_feat, cwc, g1, g2)
    lo, co = tc_heads(cell_attn, t, aggc, rcnt8[0].reshape(N_TIS, 1),
                      p["pool_cell"], p["pool_tissue"], p["fusion"],
                      p["clf"], p["count"])
    return lo[0:1], co[0:1]
